# baseline (speedup 1.0000x reference)
import jax
import jax.numpy as jnp
from jax import lax
from jax.experimental import pallas as pl
from jax.experimental.pallas import tpu as pltpu
from jax.experimental.pallas import tpu_sc as plsc

N = 10000
D = 128
E = 320000
NC, NS = 2, 16
NW = NC * NS
EPW = E // NW
K = 125
NB = EPW // K
ST = 624
TAIL = N - NS * ST
ZR = 48


HB = NB // 2


def _sc_spmm6(tables, edges, zeros):
    groups = ((0, (0, 1)), (1, (2,)), (2, (3, 4)), (3, (5,)))
    mesh = plsc.VectorSubcoreMesh(core_axis_name="c", subcore_axis_name="s")
    out_type = [jax.ShapeDtypeStruct((NC, N, D), jnp.float32) for _ in range(6)]
    scratch = [
        pltpu.VMEM_SHARED((N, D), jnp.float32),
        pltpu.VMEM((HB, K), jnp.int32),
        pltpu.VMEM((HB, K), jnp.int32),
        pltpu.VMEM((K, D), jnp.float32),
        pltpu.VMEM((K, D), jnp.float32),
        pltpu.VMEM((ZR, D), jnp.float32),
        pltpu.SemaphoreType.DMA,
    ]

    def body(*refs):
        ts = refs[0:6]
        es = refs[6:10]
        z = refs[10]
        os_ = refs[11:17]
        acc, dsti, srci, rows0, rows1, zbuf, sem_s = refs[17:24]
        rows_bufs = (rows0, rows1)
        c = lax.axis_index("c")
        s = lax.axis_index("s")
        w = c * NS + s
        pltpu.sync_copy(z, zbuf)
        for ei, tis in groups:
            e = es[ei]
            for ti in tis:
                off = pl.multiple_of(s * ST, 8)
                for jz in range(ST // ZR):
                    pltpu.sync_copy(
                        zbuf, acc.at[pl.ds(pl.multiple_of(s * ST + jz * ZR, 8), ZR)])

                @pl.when(s == NS - 1)
                def _zero_tail():
                    pltpu.sync_copy(zbuf.at[pl.ds(0, TAIL)],
                                    acc.at[pl.ds(NS * ST, TAIL)])

                plsc.subcore_barrier()
                t = ts[ti]
                for h in range(2):
                    base = w * NB + h * HB
                    pltpu.sync_copy(e.at[0, pl.ds(base, HB)], dsti)
                    pltpu.sync_copy(e.at[1, pl.ds(base, HB)], srci)
                    pltpu.sync_copy(t.at[srci.at[0]], rows0)
                    pltpu.async_copy(rows0, acc.at[dsti.at[0]], sem_s, add=True)
                    pltpu.sync_copy(t.at[srci.at[1]], rows1)
                    pltpu.async_copy(rows1, acc.at[dsti.at[1]], sem_s, add=True)

                    def bloop(i, carry, _t=t):
                        for b in range(2):
                            j = 2 * i + b
                            rb = rows_bufs[b]
                            pltpu.make_async_copy(
                                rb, acc.at[dsti.at[0]], sem_s).wait()
                            pltpu.sync_copy(_t.at[srci.at[j]], rb)
                            pltpu.async_copy(
                                rb, acc.at[dsti.at[j]], sem_s, add=True)
                        return carry

                    lax.fori_loop(1, HB // 2, bloop, 0)
                    pltpu.make_async_copy(rows0, acc.at[dsti.at[0]], sem_s).wait()
                    pltpu.make_async_copy(rows1, acc.at[dsti.at[0]], sem_s).wait()
                plsc.subcore_barrier()
                pltpu.sync_copy(acc.at[pl.ds(off, ST)],
                                os_[ti].at[c, pl.ds(off, ST)])

                @pl.when(s == NS - 1)
                def _write_tail():
                    pltpu.sync_copy(acc.at[pl.ds(NS * ST, TAIL)],
                                    os_[ti].at[c, pl.ds(NS * ST, TAIL)])

                plsc.subcore_barrier()

    return pl.kernel(body, out_type=out_type, mesh=mesh, scratch_types=scratch)(
        *tables, *edges, zeros)


_R = 1000


def _tc_matmul6(xs, W):
    def body(*refs):
        xr = refs[0:6]
        w = refs[6]
        outs = refs[7:13]
        for k in range(6):
            outs[k][...] = jnp.dot(xr[k][...], w[k],
                                   preferred_element_type=jnp.float32)

    return pl.pallas_call(
        body,
        grid=(N // _R,),
        in_specs=[pl.BlockSpec((_R, D), lambda i: (i, 0))] * 6
        + [pl.BlockSpec((6, D, D), lambda i: (0, 0, 0))],
        out_specs=[pl.BlockSpec((_R, D), lambda i: (i, 0))] * 6,
        out_shape=[jax.ShapeDtypeStruct((N, D), jnp.float32)] * 6,
    )(*xs, W)


def _tc_madd_matmul6(ps, W):
    def body(*refs):
        pr = refs[0:6]
        w = refs[6]
        outs = refs[7:13]
        for k in range(6):
            x = pr[k][0] + pr[k][1]
            outs[k][...] = jnp.dot(x, w[k], preferred_element_type=jnp.float32)

    return pl.pallas_call(
        body,
        grid=(N // _R,),
        in_specs=[pl.BlockSpec((NC, _R, D), lambda i: (0, i, 0))] * 6
        + [pl.BlockSpec((6, D, D), lambda i: (0, 0, 0))],
        out_specs=[pl.BlockSpec((_R, D), lambda i: (i, 0))] * 6,
        out_shape=[jax.ShapeDtypeStruct((N, D), jnp.float32)] * 6,
    )(*ps, W)


def _tc_final(dps, ufea_s, vfea_s, ufea_t, vfea_t, W2, B2):
    def body(*refs):
        dp = refs[0:6]
        us, vs, ut, vt, w, b = refs[6:12]
        o1, o2, o3, o4 = refs[12:16]

        def proj(k, left, right):
            wk = w[k]
            return (jnp.dot(left, wk[:D], preferred_element_type=jnp.float32)
                    + jnp.dot(right, wk[D:], preferred_element_type=jnp.float32)
                    + b[k])

        s_u = proj(0, dp[0][0] + dp[0][1], us[...])
        s_i = proj(1, dp[2][0] + dp[2][1], vs[...])
        t_u = proj(2, dp[3][0] + dp[3][1], ut[...])
        t_i = proj(3, dp[5][0] + dp[5][1], vt[...])
        share = proj(4, dp[1][0] + dp[1][1], dp[4][0] + dp[4][1])
        o1[...] = share + s_u
        o2[...] = s_i
        o3[...] = share + t_u
        o4[...] = t_i

    return pl.pallas_call(
        body,
        grid=(N // _R,),
        in_specs=[pl.BlockSpec((NC, _R, D), lambda i: (0, i, 0))] * 6
        + [pl.BlockSpec((_R, D), lambda i: (i, 0))] * 4
        + [pl.BlockSpec((5, 2 * D, D), lambda i: (0, 0, 0)),
           pl.BlockSpec((5, D), lambda i: (0, 0))],
        out_specs=[pl.BlockSpec((_R, D), lambda i: (i, 0))] * 4,
        out_shape=[jax.ShapeDtypeStruct((N, D), jnp.float32)] * 4,
    )(*dps, ufea_s, vfea_s, ufea_t, vfea_t, W2, B2)


def kernel(emb, p_src_spec, p_src_share, p_tgt_spec, p_tgt_share, p_cross,
           source_UV, source_VU, target_UV, target_VU):
    del p_src_share, p_tgt_share
    zeros = jnp.zeros((ZR, D), jnp.float32)
    e_svu = source_VU.reshape(2, E // K, K)
    e_suv = source_UV.reshape(2, E // K, K)
    e_tvu = target_VU.reshape(2, E // K, K)
    e_tuv = target_UV.reshape(2, E // K, K)

    WA = jnp.stack([p_src_spec["gc1"], p_cross["s_gc1"], p_src_spec["gc2"],
                    p_tgt_spec["gc1"], p_cross["t_gc1"], p_tgt_spec["gc2"]])
    A = _tc_matmul6([emb["source_user"], emb["source_user_share"],
                     emb["source_item"], emb["target_user"],
                     emb["target_user_share"], emb["target_item"]], WA)

    B = _sc_spmm6(A, [e_svu, e_suv, e_tvu, e_tuv], zeros)

    WC = jnp.stack([p_src_spec["gc3_mean"], p_cross["s_gc2_mean"],
                    p_src_spec["gc4_mean"], p_tgt_spec["gc3_mean"],
                    p_cross["t_gc2_mean"], p_tgt_spec["gc4_mean"]])
    C = _tc_madd_matmul6(B, WC)

    Dp = _sc_spmm6(C, [e_suv, e_svu, e_tuv, e_tvu], zeros)

    W2 = jnp.stack([p_src_spec["uu_mean_w"], p_src_spec["iu_mean_w"],
                    p_tgt_spec["uu_mean_w"], p_tgt_spec["iu_mean_w"],
                    p_cross["um_w"]])
    B2 = jnp.stack([p_src_spec["uu_mean_b"], p_src_spec["iu_mean_b"],
                    p_tgt_spec["uu_mean_b"], p_tgt_spec["iu_mean_b"],
                    p_cross["um_b"]])
    return _tc_final(Dp, emb["source_user"], emb["source_item"],
                     emb["target_user"], emb["target_item"], W2, B2)


# speedup vs baseline: 1.0041x; 1.0041x over previous
import jax
import jax.numpy as jnp
from jax import lax
from jax.experimental import pallas as pl
from jax.experimental.pallas import tpu as pltpu
from jax.experimental.pallas import tpu_sc as plsc

N = 10000
D = 128
E = 320000
NC, NS = 2, 16
NW = NC * NS
EPW = E // NW
K = 125
NB = EPW // K
ST = 624
TAIL = N - NS * ST
ZR = 48


HB = NB // 2


def _sc_spmm6(tables, edges, zeros):
    groups = ((0, (0, 1)), (1, (2,)), (2, (3, 4)), (3, (5,)))
    mesh = plsc.VectorSubcoreMesh(core_axis_name="c", subcore_axis_name="s")
    out_type = [jax.ShapeDtypeStruct((NC, N, D), jnp.float32) for _ in range(6)]
    scratch = [
        pltpu.VMEM_SHARED((N, D), jnp.float32),
        pltpu.VMEM((HB, K), jnp.int32),
        pltpu.VMEM((HB, K), jnp.int32),
        pltpu.VMEM((K, D), jnp.float32),
        pltpu.VMEM((K, D), jnp.float32),
        pltpu.VMEM((ZR, D), jnp.float32),
        pltpu.SemaphoreType.DMA,
        pltpu.SemaphoreType.DMA,
    ]

    def body(*refs):
        ts = refs[0:6]
        es = refs[6:10]
        z = refs[10]
        os_ = refs[11:17]
        acc, dsti, srci, rows0, rows1, zbuf, sem_s, sem_z = refs[17:25]
        rows_bufs = (rows0, rows1)
        c = lax.axis_index("c")
        s = lax.axis_index("s")
        w = c * NS + s
        pltpu.sync_copy(z, zbuf)
        for ei, tis in groups:
            e = es[ei]
            for ti in tis:
                off = pl.multiple_of(s * ST, 8)
                for jz in range(ST // ZR):
                    pltpu.async_copy(
                        zbuf, acc.at[pl.ds(pl.multiple_of(s * ST + jz * ZR, 8), ZR)],
                        sem_z)

                @pl.when(s == NS - 1)
                def _zero_tail():
                    pltpu.async_copy(zbuf.at[pl.ds(0, TAIL)],
                                     acc.at[pl.ds(NS * ST, TAIL)], sem_z)

                for jz in range(ST // ZR):
                    pltpu.make_async_copy(
                        zbuf, acc.at[pl.ds(pl.multiple_of(s * ST + jz * ZR, 8), ZR)],
                        sem_z).wait()

                @pl.when(s == NS - 1)
                def _zero_tail_wait():
                    pltpu.make_async_copy(zbuf.at[pl.ds(0, TAIL)],
                                          acc.at[pl.ds(NS * ST, TAIL)],
                                          sem_z).wait()

                plsc.subcore_barrier()
                t = ts[ti]
                for h in range(2):
                    base = w * NB + h * HB
                    pltpu.sync_copy(e.at[0, pl.ds(base, HB)], dsti)
                    pltpu.sync_copy(e.at[1, pl.ds(base, HB)], srci)
                    pltpu.sync_copy(t.at[srci.at[0]], rows0)
                    pltpu.async_copy(rows0, acc.at[dsti.at[0]], sem_s, add=True)
                    pltpu.sync_copy(t.at[srci.at[1]], rows1)
                    pltpu.async_copy(rows1, acc.at[dsti.at[1]], sem_s, add=True)

                    def bloop(i, carry, _t=t):
                        for b in range(2):
                            j = 2 * i + b
                            rb = rows_bufs[b]
                            pltpu.make_async_copy(
                                rb, acc.at[dsti.at[0]], sem_s).wait()
                            pltpu.sync_copy(_t.at[srci.at[j]], rb)
                            pltpu.async_copy(
                                rb, acc.at[dsti.at[j]], sem_s, add=True)
                        return carry

                    lax.fori_loop(1, HB // 2, bloop, 0)
                    pltpu.make_async_copy(rows0, acc.at[dsti.at[0]], sem_s).wait()
                    pltpu.make_async_copy(rows1, acc.at[dsti.at[0]], sem_s).wait()
                plsc.subcore_barrier()
                pltpu.sync_copy(acc.at[pl.ds(off, ST)],
                                os_[ti].at[c, pl.ds(off, ST)])

                @pl.when(s == NS - 1)
                def _write_tail():
                    pltpu.sync_copy(acc.at[pl.ds(NS * ST, TAIL)],
                                    os_[ti].at[c, pl.ds(NS * ST, TAIL)])

                plsc.subcore_barrier()

    return pl.kernel(body, out_type=out_type, mesh=mesh, scratch_types=scratch)(
        *tables, *edges, zeros)


_R = 1000


def _tc_matmul6(xs, W):
    def body(*refs):
        xr = refs[0:6]
        w = refs[6]
        outs = refs[7:13]
        for k in range(6):
            outs[k][...] = jnp.dot(xr[k][...], w[k],
                                   preferred_element_type=jnp.float32)

    return pl.pallas_call(
        body,
        grid=(N // _R,),
        in_specs=[pl.BlockSpec((_R, D), lambda i: (i, 0))] * 6
        + [pl.BlockSpec((6, D, D), lambda i: (0, 0, 0))],
        out_specs=[pl.BlockSpec((_R, D), lambda i: (i, 0))] * 6,
        out_shape=[jax.ShapeDtypeStruct((N, D), jnp.float32)] * 6,
    )(*xs, W)


def _tc_madd_matmul6(ps, W):
    def body(*refs):
        pr = refs[0:6]
        w = refs[6]
        outs = refs[7:13]
        for k in range(6):
            x = pr[k][0] + pr[k][1]
            outs[k][...] = jnp.dot(x, w[k], preferred_element_type=jnp.float32)

    return pl.pallas_call(
        body,
        grid=(N // _R,),
        in_specs=[pl.BlockSpec((NC, _R, D), lambda i: (0, i, 0))] * 6
        + [pl.BlockSpec((6, D, D), lambda i: (0, 0, 0))],
        out_specs=[pl.BlockSpec((_R, D), lambda i: (i, 0))] * 6,
        out_shape=[jax.ShapeDtypeStruct((N, D), jnp.float32)] * 6,
    )(*ps, W)


def _tc_final(dps, ufea_s, vfea_s, ufea_t, vfea_t, W2, B2):
    def body(*refs):
        dp = refs[0:6]
        us, vs, ut, vt, w, b = refs[6:12]
        o1, o2, o3, o4 = refs[12:16]

        def proj(k, left, right):
            wk = w[k]
            return (jnp.dot(left, wk[:D], preferred_element_type=jnp.float32)
                    + jnp.dot(right, wk[D:], preferred_element_type=jnp.float32)
                    + b[k])

        s_u = proj(0, dp[0][0] + dp[0][1], us[...])
        s_i = proj(1, dp[2][0] + dp[2][1], vs[...])
        t_u = proj(2, dp[3][0] + dp[3][1], ut[...])
        t_i = proj(3, dp[5][0] + dp[5][1], vt[...])
        share = proj(4, dp[1][0] + dp[1][1], dp[4][0] + dp[4][1])
        o1[...] = share + s_u
        o2[...] = s_i
        o3[...] = share + t_u
        o4[...] = t_i

    return pl.pallas_call(
        body,
        grid=(N // _R,),
        in_specs=[pl.BlockSpec((NC, _R, D), lambda i: (0, i, 0))] * 6
        + [pl.BlockSpec((_R, D), lambda i: (i, 0))] * 4
        + [pl.BlockSpec((5, 2 * D, D), lambda i: (0, 0, 0)),
           pl.BlockSpec((5, D), lambda i: (0, 0))],
        out_specs=[pl.BlockSpec((_R, D), lambda i: (i, 0))] * 4,
        out_shape=[jax.ShapeDtypeStruct((N, D), jnp.float32)] * 4,
    )(*dps, ufea_s, vfea_s, ufea_t, vfea_t, W2, B2)


def kernel(emb, p_src_spec, p_src_share, p_tgt_spec, p_tgt_share, p_cross,
           source_UV, source_VU, target_UV, target_VU):
    del p_src_share, p_tgt_share
    zeros = jnp.zeros((ZR, D), jnp.float32)
    e_svu = source_VU.reshape(2, E // K, K)
    e_suv = source_UV.reshape(2, E // K, K)
    e_tvu = target_VU.reshape(2, E // K, K)
    e_tuv = target_UV.reshape(2, E // K, K)

    WA = jnp.stack([p_src_spec["gc1"], p_cross["s_gc1"], p_src_spec["gc2"],
                    p_tgt_spec["gc1"], p_cross["t_gc1"], p_tgt_spec["gc2"]])
    A = _tc_matmul6([emb["source_user"], emb["source_user_share"],
                     emb["source_item"], emb["target_user"],
                     emb["target_user_share"], emb["target_item"]], WA)

    B = _sc_spmm6(A, [e_svu, e_suv, e_tvu, e_tuv], zeros)

    WC = jnp.stack([p_src_spec["gc3_mean"], p_cross["s_gc2_mean"],
                    p_src_spec["gc4_mean"], p_tgt_spec["gc3_mean"],
                    p_cross["t_gc2_mean"], p_tgt_spec["gc4_mean"]])
    C = _tc_madd_matmul6(B, WC)

    Dp = _sc_spmm6(C, [e_suv, e_svu, e_tuv, e_tvu], zeros)

    W2 = jnp.stack([p_src_spec["uu_mean_w"], p_src_spec["iu_mean_w"],
                    p_tgt_spec["uu_mean_w"], p_tgt_spec["iu_mean_w"],
                    p_cross["um_w"]])
    B2 = jnp.stack([p_src_spec["uu_mean_b"], p_src_spec["iu_mean_b"],
                    p_tgt_spec["uu_mean_b"], p_tgt_spec["iu_mean_b"],
                    p_cross["um_b"]])
    return _tc_final(Dp, emb["source_user"], emb["source_item"],
                     emb["target_user"], emb["target_item"], W2, B2)


# speedup vs baseline: 1.0102x; 1.0061x over previous
import jax
import jax.numpy as jnp
from jax import lax
from jax.experimental import pallas as pl
from jax.experimental.pallas import tpu as pltpu
from jax.experimental.pallas import tpu_sc as plsc

N = 10000
D = 128
E = 320000
NC, NS = 2, 16
NW = NC * NS
EPW = E // NW
K = 125
NB = EPW // K
ST = 624
TAIL = N - NS * ST
ZR = 48


HB = NB // 2


def _sc_spmm6(tables, edges, zeros):
    groups = ((0, (0, 1)), (1, (2,)), (2, (3, 4)), (3, (5,)))
    mesh = plsc.VectorSubcoreMesh(core_axis_name="c", subcore_axis_name="s")
    out_type = [jax.ShapeDtypeStruct((NC, N, D), jnp.float32) for _ in range(6)]
    scratch = [
        pltpu.VMEM_SHARED((N, D), jnp.float32),
        pltpu.VMEM((HB, K), jnp.int32),
        pltpu.VMEM((HB, K), jnp.int32),
        pltpu.VMEM((K, D), jnp.float32),
        pltpu.VMEM((K, D), jnp.float32),
        pltpu.VMEM((ZR, D), jnp.float32),
        pltpu.SemaphoreType.DMA,
        pltpu.SemaphoreType.DMA,
    ]

    def body(*refs):
        ts = refs[0:6]
        es = refs[6:10]
        z = refs[10]
        os_ = refs[11:17]
        acc, dsti, srci, rows0, rows1, zbuf, sem_s, sem_z = refs[17:25]
        rows_bufs = (rows0, rows1)
        c = lax.axis_index("c")
        s = lax.axis_index("s")
        w = c * NS + s
        pltpu.sync_copy(z, zbuf)
        for ei, tis in groups:
            e = es[ei]
            for ti in tis:
                off = pl.multiple_of(s * ST, 8)
                for jz in range(ST // ZR):
                    pltpu.async_copy(
                        zbuf, acc.at[pl.ds(pl.multiple_of(s * ST + jz * ZR, 8), ZR)],
                        sem_z)

                @pl.when(s == NS - 1)
                def _zero_tail():
                    pltpu.async_copy(zbuf.at[pl.ds(0, TAIL)],
                                     acc.at[pl.ds(NS * ST, TAIL)], sem_z)

                for jz in range(ST // ZR):
                    pltpu.make_async_copy(
                        zbuf, acc.at[pl.ds(pl.multiple_of(s * ST + jz * ZR, 8), ZR)],
                        sem_z).wait()

                @pl.when(s == NS - 1)
                def _zero_tail_wait():
                    pltpu.make_async_copy(zbuf.at[pl.ds(0, TAIL)],
                                          acc.at[pl.ds(NS * ST, TAIL)],
                                          sem_z).wait()

                plsc.subcore_barrier()
                t = ts[ti]
                for h in range(2):
                    base = w * NB + h * HB
                    pltpu.sync_copy(e.at[0, pl.ds(base, HB)], dsti)
                    pltpu.sync_copy(e.at[1, pl.ds(base, HB)], srci)
                    pltpu.sync_copy(t.at[srci.at[0]], rows0)
                    pltpu.async_copy(rows0, acc.at[dsti.at[0]], sem_s, add=True)
                    pltpu.sync_copy(t.at[srci.at[1]], rows1)
                    pltpu.async_copy(rows1, acc.at[dsti.at[1]], sem_s, add=True)

                    def bloop(i, carry, _t=t):
                        for b in range(2):
                            j = 2 * i + b
                            rb = rows_bufs[b]
                            pltpu.make_async_copy(
                                rb, acc.at[dsti.at[0]], sem_s).wait()
                            pltpu.sync_copy(_t.at[srci.at[j]], rb)
                            pltpu.async_copy(
                                rb, acc.at[dsti.at[j]], sem_s, add=True)
                        return carry

                    lax.fori_loop(1, HB // 2, bloop, 0)
                    pltpu.make_async_copy(rows0, acc.at[dsti.at[0]], sem_s).wait()
                    pltpu.make_async_copy(rows1, acc.at[dsti.at[0]], sem_s).wait()
                plsc.subcore_barrier()
                pltpu.sync_copy(acc.at[pl.ds(off, ST)],
                                os_[ti].at[c, pl.ds(off, ST)])

                @pl.when(s == NS - 1)
                def _write_tail():
                    pltpu.sync_copy(acc.at[pl.ds(NS * ST, TAIL)],
                                    os_[ti].at[c, pl.ds(NS * ST, TAIL)])

    return pl.kernel(body, out_type=out_type, mesh=mesh, scratch_types=scratch)(
        *tables, *edges, zeros)


_R = 1000


def _tc_matmul6(xs, W):
    def body(*refs):
        xr = refs[0:6]
        w = refs[6]
        outs = refs[7:13]
        for k in range(6):
            outs[k][...] = jnp.dot(xr[k][...], w[k],
                                   preferred_element_type=jnp.float32)

    return pl.pallas_call(
        body,
        grid=(N // _R,),
        in_specs=[pl.BlockSpec((_R, D), lambda i: (i, 0))] * 6
        + [pl.BlockSpec((6, D, D), lambda i: (0, 0, 0))],
        out_specs=[pl.BlockSpec((_R, D), lambda i: (i, 0))] * 6,
        out_shape=[jax.ShapeDtypeStruct((N, D), jnp.float32)] * 6,
    )(*xs, W)


def _tc_madd_matmul6(ps, W):
    def body(*refs):
        pr = refs[0:6]
        w = refs[6]
        outs = refs[7:13]
        for k in range(6):
            x = pr[k][0] + pr[k][1]
            outs[k][...] = jnp.dot(x, w[k], preferred_element_type=jnp.float32)

    return pl.pallas_call(
        body,
        grid=(N // _R,),
        in_specs=[pl.BlockSpec((NC, _R, D), lambda i: (0, i, 0))] * 6
        + [pl.BlockSpec((6, D, D), lambda i: (0, 0, 0))],
        out_specs=[pl.BlockSpec((_R, D), lambda i: (i, 0))] * 6,
        out_shape=[jax.ShapeDtypeStruct((N, D), jnp.float32)] * 6,
    )(*ps, W)


def _tc_final(dps, ufea_s, vfea_s, ufea_t, vfea_t, W2, B2):
    def body(*refs):
        dp = refs[0:6]
        us, vs, ut, vt, w, b = refs[6:12]
        o1, o2, o3, o4 = refs[12:16]

        def proj(k, left, right):
            wk = w[k]
            return (jnp.dot(left, wk[:D], preferred_element_type=jnp.float32)
                    + jnp.dot(right, wk[D:], preferred_element_type=jnp.float32)
                    + b[k])

        s_u = proj(0, dp[0][0] + dp[0][1], us[...])
        s_i = proj(1, dp[2][0] + dp[2][1], vs[...])
        t_u = proj(2, dp[3][0] + dp[3][1], ut[...])
        t_i = proj(3, dp[5][0] + dp[5][1], vt[...])
        share = proj(4, dp[1][0] + dp[1][1], dp[4][0] + dp[4][1])
        o1[...] = share + s_u
        o2[...] = s_i
        o3[...] = share + t_u
        o4[...] = t_i

    return pl.pallas_call(
        body,
        grid=(N // _R,),
        in_specs=[pl.BlockSpec((NC, _R, D), lambda i: (0, i, 0))] * 6
        + [pl.BlockSpec((_R, D), lambda i: (i, 0))] * 4
        + [pl.BlockSpec((5, 2 * D, D), lambda i: (0, 0, 0)),
           pl.BlockSpec((5, D), lambda i: (0, 0))],
        out_specs=[pl.BlockSpec((_R, D), lambda i: (i, 0))] * 4,
        out_shape=[jax.ShapeDtypeStruct((N, D), jnp.float32)] * 4,
    )(*dps, ufea_s, vfea_s, ufea_t, vfea_t, W2, B2)


def kernel(emb, p_src_spec, p_src_share, p_tgt_spec, p_tgt_share, p_cross,
           source_UV, source_VU, target_UV, target_VU):
    del p_src_share, p_tgt_share
    zeros = jnp.zeros((ZR, D), jnp.float32)
    e_svu = source_VU.reshape(2, E // K, K)
    e_suv = source_UV.reshape(2, E // K, K)
    e_tvu = target_VU.reshape(2, E // K, K)
    e_tuv = target_UV.reshape(2, E // K, K)

    WA = jnp.stack([p_src_spec["gc1"], p_cross["s_gc1"], p_src_spec["gc2"],
                    p_tgt_spec["gc1"], p_cross["t_gc1"], p_tgt_spec["gc2"]])
    A = _tc_matmul6([emb["source_user"], emb["source_user_share"],
                     emb["source_item"], emb["target_user"],
                     emb["target_user_share"], emb["target_item"]], WA)

    B = _sc_spmm6(A, [e_svu, e_suv, e_tvu, e_tuv], zeros)

    WC = jnp.stack([p_src_spec["gc3_mean"], p_cross["s_gc2_mean"],
                    p_src_spec["gc4_mean"], p_tgt_spec["gc3_mean"],
                    p_cross["t_gc2_mean"], p_tgt_spec["gc4_mean"]])
    C = _tc_madd_matmul6(B, WC)

    Dp = _sc_spmm6(C, [e_suv, e_svu, e_tuv, e_tvu], zeros)

    W2 = jnp.stack([p_src_spec["uu_mean_w"], p_src_spec["iu_mean_w"],
                    p_tgt_spec["uu_mean_w"], p_tgt_spec["iu_mean_w"],
                    p_cross["um_w"]])
    B2 = jnp.stack([p_src_spec["uu_mean_b"], p_src_spec["iu_mean_b"],
                    p_tgt_spec["uu_mean_b"], p_tgt_spec["iu_mean_b"],
                    p_cross["um_b"]])
    return _tc_final(Dp, emb["source_user"], emb["source_item"],
                     emb["target_user"], emb["target_item"], W2, B2)
